# baseline (speedup 1.0000x reference)
import functools

import jax
import jax.numpy as jnp
from jax import lax
from jax.experimental import pallas as pl
from jax.experimental.pallas import tpu as pltpu
from jax.experimental.pallas import tpu_sc as plsc

NC = 2
NS = 16
NW = NC * NS
CHUNK = 128


@functools.lru_cache(maxsize=None)
def _make_emb(n_chunks, embed_dim):
  mesh = plsc.VectorSubcoreMesh(core_axis_name="c", subcore_axis_name="s")

  @functools.partial(
      pl.kernel,
      mesh=mesh,
      compiler_params=pltpu.CompilerParams(use_tc_tiling_on_sc=False),
      out_type=jax.ShapeDtypeStruct((NW * n_chunks, CHUNK, embed_dim),
                                    jnp.float32),
      scratch_types=[
          pltpu.VMEM((n_chunks, CHUNK), jnp.int32),
          pltpu.VMEM((n_chunks, CHUNK, embed_dim), jnp.float32),
          pltpu.SemaphoreType.DMA,
      ],
  )
  def emb(idx_hbm, table_hbm, out_hbm, idx_v, rows_v, sem):
    wid = lax.axis_index("s") * NC + lax.axis_index("c")
    base = wid * n_chunks
    pltpu.sync_copy(idx_hbm.at[pl.ds(base, n_chunks)], idx_v)
    copies = [
        pltpu.async_copy(table_hbm.at[idx_v.at[j]], rows_v.at[j], sem)
        for j in range(n_chunks)
    ]
    for c in copies:
      c.wait()
    pltpu.sync_copy(rows_v, out_hbm.at[pl.ds(base, n_chunks)])

  return emb


@jax.jit
def kernel(cls_idx, table):
  b = cls_idx.shape[0]
  embed_dim = table.shape[1]
  n_chunks = b // (NW * CHUNK)
  idx = cls_idx.astype(jnp.int32).reshape(NW * n_chunks, CHUNK)
  out = _make_emb(n_chunks, embed_dim)(idx, table)
  return out.reshape(b, 1, embed_dim)


# speedup vs baseline: 3.9197x; 3.9197x over previous
import functools

import jax
import jax.numpy as jnp
from jax import lax
from jax.experimental import pallas as pl
from jax.experimental.pallas import tpu as pltpu
from jax.experimental.pallas import tpu_sc as plsc

NC = 2
NS = 16
NW = NC * NS
LANES = 16
WIN = 128


@functools.lru_cache(maxsize=None)
def _make_emb(b, embed_dim):
  mesh = plsc.VectorSubcoreMesh(core_axis_name="c", subcore_axis_name="s")
  b_per_w = b // NW
  n_groups = b_per_w // LANES
  n_rows = embed_dim // LANES

  @functools.partial(
      pl.kernel,
      mesh=mesh,
      compiler_params=pltpu.CompilerParams(needs_layout_passes=False),
      out_type=jax.ShapeDtypeStruct((embed_dim, b), jnp.float32),
      scratch_types=[
          pltpu.VMEM((b_per_w,), jnp.int32),
          pltpu.VMEM((LANES, embed_dim, WIN), jnp.float32),
          pltpu.VMEM((embed_dim, WIN), jnp.float32),
          pltpu.SemaphoreType.DMA,
      ],
  )
  def emb(idx_hbm, tbl_t_hbm, out_hbm, idx_v, win_v, dims_v, sem):
    wid = lax.axis_index("s") * NC + lax.axis_index("c")
    base = wid * b_per_w
    pltpu.sync_copy(idx_hbm.at[wid], idx_v)
    row_iota = [
        lax.iota(jnp.int32, LANES) + k * LANES for k in range(n_rows)
    ]

    def body(g):
      vec = idx_v[pl.ds(g * LANES, LANES)]
      copies = []
      for l in range(LANES):
        lane_base = pl.multiple_of((vec[l] >> 7) << 7, WIN)
        copies.append(
            pltpu.async_copy(
                tbl_t_hbm.at[:, pl.ds(lane_base, WIN)], win_v.at[l], sem))
      col_in_group = (g & 7) * LANES
      for l in range(LANES):
        copies[l].wait()
        col = jnp.broadcast_to(vec[l] & 127, (LANES,))
        pos = jnp.broadcast_to(col_in_group + l, (LANES,))
        for k in range(n_rows):
          vals = plsc.load_gather(win_v.at[l], [row_iota[k], col])
          plsc.store_scatter(dims_v, [row_iota[k], pos], vals)

      @pl.when((g & 7) == 7)
      def _flush():
        out_base = pl.multiple_of(base + ((g >> 3) << 7), WIN)
        pltpu.sync_copy(dims_v, out_hbm.at[:, pl.ds(out_base, WIN)])

    pl.loop(0, n_groups)(body)

  return emb


@jax.jit
def kernel(cls_idx, table):
  b = cls_idx.shape[0]
  embed_dim = table.shape[1]
  idx = cls_idx.astype(jnp.int32).reshape(NW, b // NW)
  out_t = _make_emb(b, embed_dim)(idx, table.T)
  return out_t.T.reshape(b, 1, embed_dim)
